# Initial kernel scaffold; baseline (speedup 1.0000x reference)
import jax
import jax.numpy as jnp
from jax.experimental import pallas as pl


def kernel(actor_feat, lane_feat, lane_centers, lane_key_valid_mask, x_centers, x_key_valid_mask, spike_rate, W1_0, b1_0, W2_0, b2_0, W1_1, b1_1, W2_1, b2_1, ln_g, ln_b):
    raise NotImplementedError("write your pallas kernel here")


# speedup vs baseline: 3.8584x; 3.8584x over previous
import functools

import jax
import jax.numpy as jnp
from jax import lax
from jax.experimental import pallas as pl
from jax.experimental.pallas import tpu as pltpu

B, NA, NL, D = 256, 256, 1024, 128
K_ACT, K_LANE = 16, 16

BB_A = 32
BB = 8
NEG_BIG = -jnp.inf
POS_BIG = jnp.inf


def _select_body(spike_ref, amask_ref, lmask_ref, ax_ref, ay_ref, lcx_ref,
                 lcy_ref, aidx_ref, lidx_ref):
    bb = BB_A
    iota_na = lax.broadcasted_iota(jnp.int32, (bb, NA), 1)
    iota_nl = lax.broadcasted_iota(jnp.int32, (bb, NL), 1)

    cur = spike_ref[...] + amask_ref[...]
    ax = ax_ref[...]
    ay = ay_ref[...]
    lcx = lcx_ref[...]
    lcy = lcy_ref[...]

    lane_d = jnp.full((bb, NL), POS_BIG, dtype=jnp.float32)
    aidx_cols = []
    for _ in range(K_ACT):
        m = jnp.max(cur, axis=1, keepdims=True)
        hit = cur == m
        idx_t = jnp.min(jnp.where(hit, iota_na, NA), axis=1, keepdims=True)
        sel = iota_na == idx_t
        acx = jnp.max(jnp.where(sel, ax, NEG_BIG), axis=1, keepdims=True)
        acy = jnp.max(jnp.where(sel, ay, NEG_BIG), axis=1, keepdims=True)
        dx = lcx - acx
        dy = lcy - acy
        lane_d = jnp.minimum(lane_d, dx * dx + dy * dy)
        cur = jnp.where(sel, NEG_BIG, cur)
        aidx_cols.append(idx_t)
    aidx_ref[...] = jnp.concatenate(aidx_cols, axis=1)

    lane_d = lane_d + lmask_ref[...]
    lidx_cols = []
    for _ in range(K_LANE):
        m = jnp.min(lane_d, axis=1, keepdims=True)
        hit = lane_d == m
        idx_t = jnp.min(jnp.where(hit, iota_nl, NL), axis=1, keepdims=True)
        sel = iota_nl == idx_t
        lane_d = jnp.where(sel, POS_BIG, lane_d)
        lidx_cols.append(idx_t)
    lidx_ref[...] = jnp.concatenate(lidx_cols, axis=1)


def _update_body(aidx_sref, lidx_sref, actor_in, lane_in,
                 W1_0, b1_0, W2_0, b2_0, W1_1, b1_1, W2_1, b2_1, ln_g, ln_b,
                 actor_out, lane_out, nodes_ref):
    g = pl.program_id(0)
    actor_out[...] = actor_in[...]
    lane_out[...] = lane_in[...]

    for b in range(BB):
        for t in range(K_ACT):
            ia = aidx_sref[g * BB + b, t]
            nodes_ref[pl.ds(b * 32 + t, 1), :] = actor_in[b, pl.ds(ia, 1), :]
        for t in range(K_LANE):
            il = lidx_sref[g * BB + b, t]
            nodes_ref[pl.ds(b * 32 + K_ACT + t, 1), :] = lane_in[b, pl.ds(il, 1), :]

    nodes = nodes_ref[...]
    params = [(W1_0, b1_0, W2_0, b2_0), (W1_1, b1_1, W2_1, b2_1)]
    gv = ln_g[...]
    bv = ln_b[...]
    for (W1, b1, W2, b2) in params:
        h = lax.dot_general(nodes, W1[...], (((1,), (1,)), ((), ())),
                            preferred_element_type=jnp.float32) + b1[...]
        h = h * 0.5 * (1.0 + lax.erf(h * 0.7071067811865476))
        h = lax.dot_general(h, W2[...], (((1,), (1,)), ((), ())),
                            preferred_element_type=jnp.float32) + b2[...]
        x = nodes + h
        mu = jnp.mean(x, axis=-1, keepdims=True)
        var = jnp.mean((x - mu) * (x - mu), axis=-1, keepdims=True)
        nodes = (x - mu) / jnp.sqrt(var + 1e-5) * gv + bv

    for b in range(BB):
        for t in range(K_ACT):
            ia = aidx_sref[g * BB + b, t]
            actor_out[b, pl.ds(ia, 1), :] = nodes[b * 32 + t][None, :]
        for t in range(K_LANE):
            il = lidx_sref[g * BB + b, t]
            lane_out[b, pl.ds(il, 1), :] = nodes[b * 32 + K_ACT + t][None, :]


def kernel(actor_feat, lane_feat, lane_centers, lane_key_valid_mask, x_centers,
           x_key_valid_mask, spike_rate, W1_0, b1_0, W2_0, b2_0, W1_1, b1_1,
           W2_1, b2_1, ln_g, ln_b):
    f32 = jnp.float32
    amask = jnp.where(x_key_valid_mask, 0.0, NEG_BIG).astype(f32)
    lmask = jnp.where(lane_key_valid_mask, 0.0, POS_BIG).astype(f32)
    ax = x_centers[:, :, 0]
    ay = x_centers[:, :, 1]
    lcx = lane_centers[:, :, 0]
    lcy = lane_centers[:, :, 1]

    na_spec = pl.BlockSpec((BB_A, NA), lambda i: (i, 0))
    nl_spec = pl.BlockSpec((BB_A, NL), lambda i: (i, 0))
    idx_spec = pl.BlockSpec((BB_A, 16), lambda i: (i, 0))
    aidx, lidx = pl.pallas_call(
        _select_body,
        grid=(B // BB_A,),
        in_specs=[na_spec, na_spec, nl_spec, na_spec, na_spec, nl_spec, nl_spec],
        out_specs=[idx_spec, idx_spec],
        out_shape=[jax.ShapeDtypeStruct((B, 16), jnp.int32),
                   jax.ShapeDtypeStruct((B, 16), jnp.int32)],
    )(spike_rate, amask, lmask, ax, ay, lcx, lcy)

    w_spec = pl.BlockSpec((D, D), lambda i, *_: (0, 0))
    v_spec = pl.BlockSpec((1, D), lambda i, *_: (0, 0))
    actor_spec = pl.BlockSpec((BB, NA, D), lambda i, *_: (i, 0, 0))
    lane_spec = pl.BlockSpec((BB, NL, D), lambda i, *_: (i, 0, 0))

    grid_spec = pltpu.PrefetchScalarGridSpec(
        num_scalar_prefetch=2,
        grid=(B // BB,),
        in_specs=[actor_spec, lane_spec,
                  w_spec, v_spec, w_spec, v_spec,
                  w_spec, v_spec, w_spec, v_spec,
                  v_spec, v_spec],
        out_specs=[actor_spec, lane_spec],
        scratch_shapes=[pltpu.VMEM((BB * 32, D), f32)],
    )
    actor_out, lane_out = pl.pallas_call(
        _update_body,
        grid_spec=grid_spec,
        out_shape=[jax.ShapeDtypeStruct((B, NA, D), f32),
                   jax.ShapeDtypeStruct((B, NL, D), f32)],
    )(aidx, lidx, actor_feat, lane_feat,
      W1_0, b1_0.reshape(1, D), W2_0, b2_0.reshape(1, D),
      W1_1, b1_1.reshape(1, D), W2_1, b2_1.reshape(1, D),
      ln_g.reshape(1, D), ln_b.reshape(1, D))
    return (actor_out, lane_out)


# speedup vs baseline: 5.3056x; 1.3751x over previous
import functools

import jax
import jax.numpy as jnp
from jax import lax
from jax.experimental import pallas as pl
from jax.experimental.pallas import tpu as pltpu

B, NA, NL, D = 256, 256, 1024, 128
K_ACT, K_LANE = 16, 16

BB_A = 256
BB = 16
NEG_BIG = -jnp.inf
POS_BIG = jnp.inf


def _select_body(spike_ref, amask_ref, lmask_ref, ax_ref, ay_ref, lcx_ref,
                 lcy_ref, aidx_ref, lidx_ref):
    bb = BB_A
    iota_na = lax.broadcasted_iota(jnp.int32, (bb, NA), 1)
    iota_nl = lax.broadcasted_iota(jnp.int32, (bb, NL), 1)

    cur = spike_ref[...] + amask_ref[...]
    ax = ax_ref[...]
    ay = ay_ref[...]
    lcx = lcx_ref[...]
    lcy = lcy_ref[...]

    lane_d = jnp.full((bb, NL), POS_BIG, dtype=jnp.float32)
    aidx_cols = []
    for _ in range(K_ACT):
        m = jnp.max(cur, axis=1, keepdims=True)
        hit = cur == m
        idx_t = jnp.min(jnp.where(hit, iota_na, NA), axis=1, keepdims=True)
        sel = iota_na == idx_t
        acx = jnp.max(jnp.where(sel, ax, NEG_BIG), axis=1, keepdims=True)
        acy = jnp.max(jnp.where(sel, ay, NEG_BIG), axis=1, keepdims=True)
        dx = lcx - acx
        dy = lcy - acy
        lane_d = jnp.minimum(lane_d, dx * dx + dy * dy)
        cur = jnp.where(sel, NEG_BIG, cur)
        aidx_cols.append(idx_t)
    aidx_ref[...] = jnp.concatenate(aidx_cols, axis=1)

    lane_d = lane_d + lmask_ref[...]
    lidx_cols = []
    for _ in range(K_LANE):
        m = jnp.min(lane_d, axis=1, keepdims=True)
        hit = lane_d == m
        idx_t = jnp.min(jnp.where(hit, iota_nl, NL), axis=1, keepdims=True)
        sel = iota_nl == idx_t
        lane_d = jnp.where(sel, POS_BIG, lane_d)
        lidx_cols.append(idx_t)
    lidx_ref[...] = jnp.concatenate(lidx_cols, axis=1)


def _update_body(aidx_sref, lidx_sref, actor_in, lane_in,
                 W1_0, b1_0, W2_0, b2_0, W1_1, b1_1, W2_1, b2_1, ln_g, ln_b,
                 actor_out, lane_out, nodes_ref):
    g = pl.program_id(0)
    actor_out[...] = actor_in[...]
    lane_out[...] = lane_in[...]

    for b in range(BB):
        for t in range(K_ACT):
            ia = aidx_sref[g * BB + b, t]
            nodes_ref[pl.ds(b * 32 + t, 1), :] = actor_in[b, pl.ds(ia, 1), :]
        for t in range(K_LANE):
            il = lidx_sref[g * BB + b, t]
            nodes_ref[pl.ds(b * 32 + K_ACT + t, 1), :] = lane_in[b, pl.ds(il, 1), :]

    nodes = nodes_ref[...]
    params = [(W1_0, b1_0, W2_0, b2_0), (W1_1, b1_1, W2_1, b2_1)]
    gv = ln_g[...]
    bv = ln_b[...]
    for (W1, b1, W2, b2) in params:
        h = lax.dot_general(nodes, W1[...], (((1,), (1,)), ((), ())),
                            preferred_element_type=jnp.float32) + b1[...]
        h = h * 0.5 * (1.0 + lax.erf(h * 0.7071067811865476))
        h = lax.dot_general(h, W2[...], (((1,), (1,)), ((), ())),
                            preferred_element_type=jnp.float32) + b2[...]
        x = nodes + h
        mu = jnp.mean(x, axis=-1, keepdims=True)
        var = jnp.mean((x - mu) * (x - mu), axis=-1, keepdims=True)
        nodes = (x - mu) / jnp.sqrt(var + 1e-5) * gv + bv

    for b in range(BB):
        for t in range(K_ACT):
            ia = aidx_sref[g * BB + b, t]
            actor_out[b, pl.ds(ia, 1), :] = nodes[b * 32 + t][None, :]
        for t in range(K_LANE):
            il = lidx_sref[g * BB + b, t]
            lane_out[b, pl.ds(il, 1), :] = nodes[b * 32 + K_ACT + t][None, :]


def kernel(actor_feat, lane_feat, lane_centers, lane_key_valid_mask, x_centers,
           x_key_valid_mask, spike_rate, W1_0, b1_0, W2_0, b2_0, W1_1, b1_1,
           W2_1, b2_1, ln_g, ln_b):
    f32 = jnp.float32
    amask = jnp.where(x_key_valid_mask, 0.0, NEG_BIG).astype(f32)
    lmask = jnp.where(lane_key_valid_mask, 0.0, POS_BIG).astype(f32)
    ax = x_centers[:, :, 0]
    ay = x_centers[:, :, 1]
    lcx = lane_centers[:, :, 0]
    lcy = lane_centers[:, :, 1]

    na_spec = pl.BlockSpec((BB_A, NA), lambda i: (i, 0))
    nl_spec = pl.BlockSpec((BB_A, NL), lambda i: (i, 0))
    idx_spec = pl.BlockSpec((BB_A, 16), lambda i: (i, 0))
    aidx, lidx = pl.pallas_call(
        _select_body,
        grid=(B // BB_A,),
        in_specs=[na_spec, na_spec, nl_spec, na_spec, na_spec, nl_spec, nl_spec],
        out_specs=[idx_spec, idx_spec],
        out_shape=[jax.ShapeDtypeStruct((B, 16), jnp.int32),
                   jax.ShapeDtypeStruct((B, 16), jnp.int32)],
    )(spike_rate, amask, lmask, ax, ay, lcx, lcy)

    w_spec = pl.BlockSpec((D, D), lambda i, *_: (0, 0))
    v_spec = pl.BlockSpec((1, D), lambda i, *_: (0, 0))
    actor_spec = pl.BlockSpec((BB, NA, D), lambda i, *_: (i, 0, 0))
    lane_spec = pl.BlockSpec((BB, NL, D), lambda i, *_: (i, 0, 0))

    grid_spec = pltpu.PrefetchScalarGridSpec(
        num_scalar_prefetch=2,
        grid=(B // BB,),
        in_specs=[actor_spec, lane_spec,
                  w_spec, v_spec, w_spec, v_spec,
                  w_spec, v_spec, w_spec, v_spec,
                  v_spec, v_spec],
        out_specs=[actor_spec, lane_spec],
        scratch_shapes=[pltpu.VMEM((BB * 32, D), f32)],
    )
    actor_out, lane_out = pl.pallas_call(
        _update_body,
        grid_spec=grid_spec,
        out_shape=[jax.ShapeDtypeStruct((B, NA, D), f32),
                   jax.ShapeDtypeStruct((B, NL, D), f32)],
    )(aidx, lidx, actor_feat, lane_feat,
      W1_0, b1_0.reshape(1, D), W2_0, b2_0.reshape(1, D),
      W1_1, b1_1.reshape(1, D), W2_1, b2_1.reshape(1, D),
      ln_g.reshape(1, D), ln_b.reshape(1, D))
    return (actor_out, lane_out)
